# baseline (speedup 1.0000x reference)
import functools

import jax
import jax.numpy as jnp
from jax import lax
from jax.experimental import pallas as pl
from jax.experimental.pallas import tpu as pltpu
from jax.experimental.pallas import tpu_sc as plsc

B = 4096
L = 50
MAX_SEQ_LEN = 128
VOCAB = 10
PAD_TOKEN = 9
EMBED_DIM = 128
NBUF = 6


def kernel(player_idxs, table):
    idx_dtype = player_idxs.dtype
    info = plsc.get_sparse_core_info()
    nc, ns = info.num_cores, info.num_subcores
    nw = nc * ns
    rpw = B // nw

    mesh = plsc.VectorSubcoreMesh(core_axis_name="c", subcore_axis_name="s")

    @functools.partial(
        pl.kernel,
        mesh=mesh,
        out_type=[
            jax.ShapeDtypeStruct((B, MAX_SEQ_LEN), idx_dtype),
            jax.ShapeDtypeStruct((B, MAX_SEQ_LEN, EMBED_DIM), jnp.float32),
        ],
        scratch_types=[
            pltpu.VMEM((rpw * L + 16,), jnp.int32),
            pltpu.VMEM((rpw, MAX_SEQ_LEN), jnp.int32),
            pltpu.VMEM_SHARED((VOCAB, EMBED_DIM), jnp.float32),
            pltpu.VMEM((NBUF, MAX_SEQ_LEN, EMBED_DIM), jnp.float32),
            pltpu.SemaphoreType.DMA,
            pltpu.SemaphoreType.DMA,
        ],
    )
    def k(player_hbm, table_hbm, idxs_hbm, emb_hbm, raw_v, idx_v, table_v,
          bufs, gsem, ssem):
        wid = lax.axis_index("s") * nc + lax.axis_index("c")
        base = wid * rpw

        @pl.when(lax.axis_index("s") == 0)
        def _():
            pltpu.sync_copy(table_hbm, table_v)

        plsc.subcore_barrier()
        pltpu.sync_copy(
            player_hbm.at[pl.ds(base * L, rpw * L)], raw_v.at[pl.ds(0, rpw * L)]
        )

        pad_vec = jnp.full((16,), PAD_TOKEN, jnp.int32)
        col = lax.iota(jnp.int32, 16)
        keep = col < (L - 48)

        def pad_row(r):
            off = r * L
            for cb in range(3):
                idx_v[r, pl.ds(cb * 16, 16)] = raw_v[pl.ds(off + cb * 16, 16)]
            blk = raw_v[pl.ds(off + 48, 16)]
            idx_v[r, pl.ds(48, 16)] = jnp.where(keep, blk, pad_vec)
            for cb in range(4, 8):
                idx_v[r, pl.ds(cb * 16, 16)] = pad_vec

        niter = rpw

        def g_desc(i):
            return pltpu.make_async_copy(
                table_v.at[idx_v.at[i]], bufs.at[i % NBUF], gsem
            )

        def s_desc(i):
            return pltpu.make_async_copy(
                bufs.at[i % NBUF], emb_hbm.at[base + i], ssem,
            )

        for i in range(NBUF - 1):
            pad_row(i)
            g_desc(i).start()

        def step(i):
            @pl.when(i + NBUF - 1 < niter)
            def _():
                pad_row(i + NBUF - 1)

                @pl.when(i >= 1)
                def _():
                    s_desc(i - 1).wait()

                g_desc(i + NBUF - 1).start()

            g_desc(i).wait()
            s_desc(i).start()

        def body(i, carry):
            step(2 * i)
            step(2 * i + 1)
            return carry

        lax.fori_loop(0, niter // 2, body, 0)

        idx_out = pltpu.make_async_copy(
            idx_v, idxs_hbm.at[pl.ds(base, rpw), :], gsem
        )
        idx_out.start()
        for i in range(NBUF, 0, -1):
            s_desc(niter - i).wait()
        idx_out.wait()

    idxs, emb = k(player_idxs.reshape(-1), table)
    return (idxs.astype(idx_dtype), emb)


# speedup vs baseline: 1.6118x; 1.6118x over previous
import functools

import jax
import jax.numpy as jnp
from jax import lax
from jax.experimental import pallas as pl
from jax.experimental.pallas import tpu as pltpu
from jax.experimental.pallas import tpu_sc as plsc

B = 4096
L = 50
MAX_SEQ_LEN = 128
VOCAB = 10
PAD_TOKEN = 9
EMBED_DIM = 128
NBUF = 6


def kernel(player_idxs, table):
    idx_dtype = player_idxs.dtype
    info = plsc.get_sparse_core_info()
    nc, ns = info.num_cores, info.num_subcores
    nw = nc * ns
    rpw = B // nw

    mesh = plsc.VectorSubcoreMesh(core_axis_name="c", subcore_axis_name="s")

    @functools.partial(
        pl.kernel,
        mesh=mesh,
        out_type=[
            jax.ShapeDtypeStruct((B, MAX_SEQ_LEN), idx_dtype),
            jax.ShapeDtypeStruct((B, MAX_SEQ_LEN, EMBED_DIM), jnp.float32),
        ],
        scratch_types=[
            pltpu.VMEM((rpw * L + 16,), jnp.int32),
            pltpu.VMEM((rpw, MAX_SEQ_LEN), jnp.int32),
            pltpu.VMEM_SHARED((VOCAB, EMBED_DIM), jnp.float32),
            pltpu.VMEM((NBUF, MAX_SEQ_LEN, EMBED_DIM), jnp.float32),
            pltpu.SemaphoreType.DMA,
            pltpu.SemaphoreType.DMA,
        ],
    )
    def k(player_hbm, table_hbm, idxs_hbm, emb_hbm, raw_v, idx_v, table_v,
          bufs, gsem, ssem):
        wid = lax.axis_index("s") * nc + lax.axis_index("c")
        base = wid * rpw

        @pl.when(lax.axis_index("s") == 0)
        def _():
            pltpu.sync_copy(table_hbm, table_v)

        plsc.subcore_barrier()
        pltpu.sync_copy(
            player_hbm.at[pl.ds(base * L, rpw * L)], raw_v.at[pl.ds(0, rpw * L)]
        )

        pad_vec = jnp.full((16,), PAD_TOKEN, jnp.int32)
        col = lax.iota(jnp.int32, 16)
        keep = col < (L - 48)

        def pad_row(r):
            off = r * L
            for cb in range(3):
                idx_v[r, pl.ds(cb * 16, 16)] = raw_v[pl.ds(off + cb * 16, 16)]
            blk = raw_v[pl.ds(off + 48, 16)]
            idx_v[r, pl.ds(48, 16)] = jnp.where(keep, blk, pad_vec)
            for cb in range(4, 8):
                idx_v[r, pl.ds(cb * 16, 16)] = pad_vec

        niter = rpw

        def g_desc(i):
            return pltpu.make_async_copy(
                table_v.at[idx_v.at[i, pl.ds(0, L)]],
                bufs.at[i % NBUF, pl.ds(0, L)],
                gsem,
            )

        def s_desc(i):
            return pltpu.make_async_copy(
                bufs.at[i % NBUF], emb_hbm.at[base + i], ssem,
            )

        pad_row(0)
        tail = MAX_SEQ_LEN - L
        for b in range(NBUF):
            pltpu.make_async_copy(
                table_v.at[idx_v.at[0, pl.ds(L, tail)]],
                bufs.at[b, pl.ds(L, tail)],
                gsem,
            ).start()
        for b in range(NBUF):
            pltpu.make_async_copy(
                table_v.at[idx_v.at[0, pl.ds(L, tail)]],
                bufs.at[b, pl.ds(L, tail)],
                gsem,
            ).wait()

        for i in range(NBUF - 1):
            if i > 0:
                pad_row(i)
            g_desc(i).start()

        def step(i):
            @pl.when(i + NBUF - 1 < niter)
            def _():
                pad_row(i + NBUF - 1)

                @pl.when(i >= 1)
                def _():
                    s_desc(i - 1).wait()

                g_desc(i + NBUF - 1).start()

            g_desc(i).wait()
            s_desc(i).start()

        def body(i, carry):
            step(2 * i)
            step(2 * i + 1)
            return carry

        lax.fori_loop(0, niter // 2, body, 0)

        idx_out = pltpu.make_async_copy(
            idx_v, idxs_hbm.at[pl.ds(base, rpw), :], gsem
        )
        idx_out.start()
        for i in range(NBUF, 0, -1):
            s_desc(niter - i).wait()
        idx_out.wait()

    idxs, emb = k(player_idxs.reshape(-1), table)
    return (idxs.astype(idx_dtype), emb)
